# Initial kernel scaffold; baseline (speedup 1.0000x reference)
import functools

import jax
import jax.numpy as jnp
from jax import lax
from jax.experimental import pallas as pl
from jax.experimental.pallas import tpu as pltpu
from jax.experimental.pallas import tpu_sc as plsc

N = 10000
E = 320000
D = 128
NP = 10240
NC = 2
NS = 16
NW = NC * NS
CH = 128
NCH = 80
EPW = NCH * CH
EPAD = NW * EPW
RPT = NP // NS

_MESH = plsc.VectorSubcoreMesh(core_axis_name="c", subcore_axis_name="s")


@functools.partial(
    pl.kernel,
    out_type=jax.ShapeDtypeStruct((2, NW, NP), jnp.float32),
    mesh=_MESH,
    scratch_types=[
        pltpu.VMEM((NCH, CH), jnp.int32),
        pltpu.VMEM((NCH, CH), jnp.int32),
        pltpu.VMEM((2, NP), jnp.float32),
    ],
)
def _deg_pass(src_hbm, dst_hbm, hist_hbm, sidx, didx, hist):
    c = lax.axis_index("c")
    s = lax.axis_index("s")
    wid = s * NC + c
    pltpu.sync_copy(src_hbm.at[wid], sidx)
    pltpu.sync_copy(dst_hbm.at[wid], didx)

    zeros = jnp.zeros((16,), jnp.float32)

    def zero_body(i, carry):
        hist[0, pl.ds(i * 16, 16)] = zeros
        hist[1, pl.ds(i * 16, 16)] = zeros
        return carry

    lax.fori_loop(0, NP // 16, zero_body, 0)

    ones = jnp.full((16,), 1.0, jnp.float32)

    def body(j, carry):
        for k in range(CH // 16):
            sv = sidx[j, pl.ds(k * 16, 16)]
            dv = didx[j, pl.ds(k * 16, 16)]
            plsc.addupdate_scatter(hist.at[0], [sv], ones)
            plsc.addupdate_scatter(hist.at[1], [dv], ones)
        return carry

    lax.fori_loop(0, NCH, body, 0)
    pltpu.sync_copy(hist.at[0], hist_hbm.at[0, wid])
    pltpu.sync_copy(hist.at[1], hist_hbm.at[1, wid])


def _reduce_body(hist_ref, deg_ref):
    deg_ref[...] = jnp.sum(hist_ref[...], axis=1)


def _reduce_hist(hist):
    return pl.pallas_call(
        _reduce_body,
        out_shape=jax.ShapeDtypeStruct((2, NP), jnp.float32),
    )(hist)


def _scale_body(feat_ref, deg_ref, type_ref, wt_ref, table_ref):
    nl = lax.rsqrt(jnp.maximum(deg_ref[...], 1.0))
    tc = type_ref[...]
    lanes = lax.broadcasted_iota(jnp.int32, (tc.shape[0], 128), 1)
    wtn = jnp.sum(
        jnp.where(tc == lanes, wt_ref[...], 0.0), axis=1, keepdims=True
    )
    table_ref[...] = feat_ref[...] * (nl * wtn)


def _scale_pass(feat_p, deg_out_col, type_col, wt_row):
    blk = NP // 8
    return pl.pallas_call(
        _scale_body,
        grid=(8,),
        in_specs=[
            pl.BlockSpec((blk, D), lambda i: (i, 0)),
            pl.BlockSpec((blk, 1), lambda i: (i, 0)),
            pl.BlockSpec((blk, 1), lambda i: (i, 0)),
            pl.BlockSpec((1, 128), lambda i: (0, 0)),
        ],
        out_specs=pl.BlockSpec((blk, D), lambda i: (i, 0)),
        out_shape=jax.ShapeDtypeStruct((NP, D), jnp.float32),
    )(feat_p, deg_out_col, type_col, wt_row)


@functools.partial(
    pl.kernel,
    out_type=jax.ShapeDtypeStruct((NC, NP, D), jnp.float32),
    mesh=_MESH,
    scratch_types=[
        pltpu.VMEM((NCH, CH), jnp.int32),
        pltpu.VMEM((NCH, CH), jnp.int32),
        pltpu.VMEM((CH, D), jnp.float32),
        pltpu.VMEM_SHARED((NP, D), jnp.float32),
        pltpu.SemaphoreType.DMA,
    ],
)
def _agg_pass(table_hbm, src_hbm, dst_hbm, zero_hbm, part_hbm,
              sidx, didx, rows, acc, sem):
    c = lax.axis_index("c")
    s = lax.axis_index("s")
    wid = s * NC + c
    pltpu.sync_copy(zero_hbm, acc.at[pl.ds(s * RPT, RPT)])
    pltpu.sync_copy(src_hbm.at[wid], sidx)
    pltpu.sync_copy(dst_hbm.at[wid], didx)
    plsc.subcore_barrier()

    def body(j, carry):
        pltpu.async_copy(table_hbm.at[sidx.at[j]], rows, sem).wait()
        pltpu.sync_copy(rows, acc.at[didx.at[j]], add=True)
        return carry

    lax.fori_loop(0, NCH, body, 0)
    plsc.subcore_barrier()
    pltpu.sync_copy(acc.at[pl.ds(s * RPT, RPT)],
                    part_hbm.at[c, pl.ds(s * RPT, RPT)])


def _final_body(p_ref, w_ref, deg_ref, b_ref, o_ref):
    a = p_ref[0] + p_ref[1]
    nr = lax.rsqrt(jnp.maximum(deg_ref[...], 1.0))
    o_ref[...] = (
        jnp.dot(a * nr, w_ref[...], preferred_element_type=jnp.float32,
                precision=lax.Precision.HIGHEST)
        + b_ref[...]
    )


def _final_pass(parts, W, deg_in_col, bias_row):
    blk = NP // 8
    return pl.pallas_call(
        _final_body,
        grid=(8,),
        in_specs=[
            pl.BlockSpec((NC, blk, D), lambda i: (0, i, 0)),
            pl.BlockSpec((D, D), lambda i: (0, 0)),
            pl.BlockSpec((blk, 1), lambda i: (i, 0)),
            pl.BlockSpec((1, D), lambda i: (0, 0)),
        ],
        out_specs=pl.BlockSpec((blk, D), lambda i: (i, 0)),
        out_shape=jax.ShapeDtypeStruct((NP, D), jnp.float32),
    )(parts, W, deg_in_col, bias_row)


def kernel(feat, edge_index, type_info, W, bias, weight_type):
    src = edge_index[0].astype(jnp.int32)
    dst = edge_index[1].astype(jnp.int32)
    pad = jnp.full((EPAD - E,), N, jnp.int32)
    src3 = jnp.concatenate([src, pad]).reshape(NW, NCH, CH)
    dst3 = jnp.concatenate([dst, pad]).reshape(NW, NCH, CH)

    feat_p = jnp.zeros((NP, D), jnp.float32).at[:N].set(feat)
    type_col = (
        jnp.zeros((NP,), jnp.int32).at[:N].set(type_info.astype(jnp.int32))
        .reshape(NP, 1)
    )
    wt_row = jnp.zeros((1, 128), jnp.float32).at[0, :4].set(weight_type)
    zero_blk = jnp.zeros((RPT, D), jnp.float32)

    hist = _deg_pass(src3, dst3)
    degs = _reduce_hist(hist)
    deg_out_col = degs[0].reshape(NP, 1)
    deg_in_col = degs[1].reshape(NP, 1)

    table = _scale_pass(feat_p, deg_out_col, type_col, wt_row)
    parts = _agg_pass(table, src3, dst3, zero_blk)
    out_p = _final_pass(parts, W, deg_in_col, bias.reshape(1, D))
    return out_p[:N]


# speedup vs baseline: 3.9323x; 3.9323x over previous
import functools

import jax
import jax.numpy as jnp
from jax import lax
from jax.experimental import pallas as pl
from jax.experimental.pallas import tpu as pltpu
from jax.experimental.pallas import tpu_sc as plsc

N = 10000
E = 320000
D = 128
NP = 10240
NC = 2
NS = 16
NW = NC * NS
CH = 128
NCH = 80
EPW = NCH * CH
EPAD = NW * EPW
RPT = NP // NS

_MESH = plsc.VectorSubcoreMesh(core_axis_name="c", subcore_axis_name="s")
_SC_PARAMS = pltpu.CompilerParams(needs_layout_passes=False)


@functools.partial(
    pl.kernel,
    out_type=jax.ShapeDtypeStruct((2, NW, NP), jnp.float32),
    mesh=_MESH,
    compiler_params=_SC_PARAMS,
    scratch_types=[
        pltpu.VMEM((NCH, CH), jnp.int32),
        pltpu.VMEM((NCH, CH), jnp.int32),
        pltpu.VMEM((NP,), jnp.float32),
        pltpu.VMEM((NP,), jnp.float32),
    ],
)
def _deg_pass(src_hbm, dst_hbm, hist_hbm, sidx, didx, hs, hd):
    c = lax.axis_index("c")
    s = lax.axis_index("s")
    wid = s * NC + c
    pltpu.sync_copy(src_hbm.at[wid], sidx)
    pltpu.sync_copy(dst_hbm.at[wid], didx)

    zeros = jnp.zeros((16,), jnp.float32)

    def zero_body(i, carry):
        hs[pl.ds(i * 16, 16)] = zeros
        hd[pl.ds(i * 16, 16)] = zeros
        return carry

    lax.fori_loop(0, NP // 16, zero_body, 0)

    ones = jnp.full((16,), 1.0, jnp.float32)

    def body(j, carry):
        for k in range(CH // 16):
            sv = sidx[j, pl.ds(k * 16, 16)]
            dv = didx[j, pl.ds(k * 16, 16)]
            plsc.addupdate_scatter(hs, [sv], ones)
            plsc.addupdate_scatter(hd, [dv], ones)
        return carry

    lax.fori_loop(0, NCH, body, 0)
    pltpu.sync_copy(hs, hist_hbm.at[0, wid])
    pltpu.sync_copy(hd, hist_hbm.at[1, wid])


def _reduce_body(hist_ref, deg_ref):
    deg_ref[...] = jnp.sum(hist_ref[...], axis=1)


def _reduce_hist(hist):
    return pl.pallas_call(
        _reduce_body,
        out_shape=jax.ShapeDtypeStruct((2, NP), jnp.float32),
    )(hist)


def _scale_body(feat_ref, deg_ref, type_ref, wt_ref, table_ref):
    nl = lax.rsqrt(jnp.maximum(deg_ref[...], 1.0))
    tc = type_ref[...]
    lanes = lax.broadcasted_iota(jnp.int32, (tc.shape[0], 128), 1)
    wtn = jnp.sum(
        jnp.where(tc == lanes, wt_ref[...], 0.0), axis=1, keepdims=True
    )
    table_ref[...] = feat_ref[...] * (nl * wtn)


def _scale_pass(feat_p, deg_out_col, type_col, wt_row):
    blk = NP // 8
    return pl.pallas_call(
        _scale_body,
        grid=(8,),
        in_specs=[
            pl.BlockSpec((blk, D), lambda i: (i, 0)),
            pl.BlockSpec((blk, 1), lambda i: (i, 0)),
            pl.BlockSpec((blk, 1), lambda i: (i, 0)),
            pl.BlockSpec((1, 128), lambda i: (0, 0)),
        ],
        out_specs=pl.BlockSpec((blk, D), lambda i: (i, 0)),
        out_shape=jax.ShapeDtypeStruct((NP, D), jnp.float32),
    )(feat_p, deg_out_col, type_col, wt_row)


@functools.partial(
    pl.kernel,
    out_type=jax.ShapeDtypeStruct((NC, NP, D), jnp.float32),
    mesh=_MESH,
    compiler_params=_SC_PARAMS,
    scratch_types=[
        pltpu.VMEM((NCH, CH), jnp.int32),
        pltpu.VMEM((NCH, CH), jnp.int32),
        pltpu.VMEM((CH, D), jnp.float32),
        pltpu.VMEM_SHARED((NP, D), jnp.float32),
        pltpu.SemaphoreType.DMA,
    ],
)
def _agg_pass(table_hbm, src_hbm, dst_hbm, zero_hbm, part_hbm,
              sidx, didx, rows, acc, sem):
    c = lax.axis_index("c")
    s = lax.axis_index("s")
    wid = s * NC + c
    pltpu.sync_copy(zero_hbm, acc.at[pl.ds(s * RPT, RPT)])
    pltpu.sync_copy(src_hbm.at[wid], sidx)
    pltpu.sync_copy(dst_hbm.at[wid], didx)
    plsc.subcore_barrier()

    def body(j, carry):
        pltpu.async_copy(table_hbm.at[sidx.at[j]], rows, sem).wait()
        pltpu.sync_copy(rows, acc.at[didx.at[j]], add=True)
        return carry

    lax.fori_loop(0, NCH, body, 0)
    plsc.subcore_barrier()
    pltpu.sync_copy(acc.at[pl.ds(s * RPT, RPT)],
                    part_hbm.at[c, pl.ds(s * RPT, RPT)])


def _final_body(p_ref, w_ref, deg_ref, b_ref, o_ref):
    a = p_ref[0] + p_ref[1]
    nr = lax.rsqrt(jnp.maximum(deg_ref[...], 1.0))
    o_ref[...] = (
        jnp.dot(a * nr, w_ref[...], preferred_element_type=jnp.float32,
                precision=lax.Precision.HIGHEST)
        + b_ref[...]
    )


def _final_pass(parts, W, deg_in_col, bias_row):
    blk = NP // 8
    return pl.pallas_call(
        _final_body,
        grid=(8,),
        in_specs=[
            pl.BlockSpec((NC, blk, D), lambda i: (0, i, 0)),
            pl.BlockSpec((D, D), lambda i: (0, 0)),
            pl.BlockSpec((blk, 1), lambda i: (i, 0)),
            pl.BlockSpec((1, D), lambda i: (0, 0)),
        ],
        out_specs=pl.BlockSpec((blk, D), lambda i: (i, 0)),
        out_shape=jax.ShapeDtypeStruct((NP, D), jnp.float32),
    )(parts, W, deg_in_col, bias_row)


def kernel(feat, edge_index, type_info, W, bias, weight_type):
    src = edge_index[0].astype(jnp.int32)
    dst = edge_index[1].astype(jnp.int32)
    pad = jnp.full((EPAD - E,), N, jnp.int32)
    src3 = jnp.concatenate([src, pad]).reshape(NW, NCH, CH)
    dst3 = jnp.concatenate([dst, pad]).reshape(NW, NCH, CH)

    feat_p = jnp.zeros((NP, D), jnp.float32).at[:N].set(feat)
    type_col = (
        jnp.zeros((NP,), jnp.int32).at[:N].set(type_info.astype(jnp.int32))
        .reshape(NP, 1)
    )
    wt_row = jnp.zeros((1, 128), jnp.float32).at[0, :4].set(weight_type)
    zero_blk = jnp.zeros((RPT, D), jnp.float32)

    hist = _deg_pass(src3, dst3)
    degs = _reduce_hist(hist)
    deg_out_col = degs[0].reshape(NP, 1)
    deg_in_col = degs[1].reshape(NP, 1)

    table = _scale_pass(feat_p, deg_out_col, type_col, wt_row)
    parts = _agg_pass(table, src3, dst3, zero_blk)
    out_p = _final_pass(parts, W, deg_in_col, bias.reshape(1, D))
    return out_p[:N]


# speedup vs baseline: 4.2037x; 1.0690x over previous
import functools

import jax
import jax.numpy as jnp
from jax import lax
from jax.experimental import pallas as pl
from jax.experimental.pallas import tpu as pltpu
from jax.experimental.pallas import tpu_sc as plsc

N = 10000
E = 320000
D = 128
NP = 10240
NC = 2
NS = 16
NW = NC * NS
CH = 128
NCH = 80
EPW = NCH * CH
EPAD = NW * EPW
RPT = NP // NS
GRP = 8

_MESH = plsc.VectorSubcoreMesh(core_axis_name="c", subcore_axis_name="s")
_SC_PARAMS = pltpu.CompilerParams(needs_layout_passes=False)


@functools.partial(
    pl.kernel,
    out_type=jax.ShapeDtypeStruct((2, NW, NP), jnp.float32),
    mesh=_MESH,
    compiler_params=_SC_PARAMS,
    scratch_types=[
        pltpu.VMEM((NCH, CH), jnp.int32),
        pltpu.VMEM((NCH, CH), jnp.int32),
        pltpu.VMEM((NP,), jnp.float32),
        pltpu.VMEM((NP,), jnp.float32),
    ],
)
def _deg_pass(src_hbm, dst_hbm, hist_hbm, sidx, didx, hs, hd):
    c = lax.axis_index("c")
    s = lax.axis_index("s")
    wid = s * NC + c
    pltpu.sync_copy(src_hbm.at[wid], sidx)
    pltpu.sync_copy(dst_hbm.at[wid], didx)

    zeros = jnp.zeros((16,), jnp.float32)

    def zero_body(i, carry):
        hs[pl.ds(i * 16, 16)] = zeros
        hd[pl.ds(i * 16, 16)] = zeros
        return carry

    lax.fori_loop(0, NP // 16, zero_body, 0)

    ones = jnp.full((16,), 1.0, jnp.float32)

    def body(j, carry):
        for k in range(CH // 16):
            sv = sidx[j, pl.ds(k * 16, 16)]
            dv = didx[j, pl.ds(k * 16, 16)]
            plsc.addupdate_scatter(hs, [sv], ones)
            plsc.addupdate_scatter(hd, [dv], ones)
        return carry

    lax.fori_loop(0, NCH, body, 0)
    pltpu.sync_copy(hs, hist_hbm.at[0, wid])
    pltpu.sync_copy(hd, hist_hbm.at[1, wid])


def _reduce_body(hist_ref, deg_ref):
    deg_ref[...] = jnp.sum(hist_ref[...], axis=1)


def _reduce_hist(hist):
    return pl.pallas_call(
        _reduce_body,
        out_shape=jax.ShapeDtypeStruct((2, NP), jnp.float32),
    )(hist)


def _scale_body(feat_ref, deg_ref, type_ref, wt_ref, table_ref):
    nl = lax.rsqrt(jnp.maximum(deg_ref[...], 1.0))
    tc = type_ref[...]
    lanes = lax.broadcasted_iota(jnp.int32, (tc.shape[0], 128), 1)
    wtn = jnp.sum(
        jnp.where(tc == lanes, wt_ref[...], 0.0), axis=1, keepdims=True
    )
    table_ref[...] = feat_ref[...] * (nl * wtn)


def _scale_pass(feat_p, deg_out_col, type_col, wt_row):
    blk = NP // 8
    return pl.pallas_call(
        _scale_body,
        grid=(8,),
        in_specs=[
            pl.BlockSpec((blk, D), lambda i: (i, 0)),
            pl.BlockSpec((blk, 1), lambda i: (i, 0)),
            pl.BlockSpec((blk, 1), lambda i: (i, 0)),
            pl.BlockSpec((1, 128), lambda i: (0, 0)),
        ],
        out_specs=pl.BlockSpec((blk, D), lambda i: (i, 0)),
        out_shape=jax.ShapeDtypeStruct((NP, D), jnp.float32),
    )(feat_p, deg_out_col, type_col, wt_row)


@functools.partial(
    pl.kernel,
    out_type=jax.ShapeDtypeStruct((NC, NP, D), jnp.float32),
    mesh=_MESH,
    compiler_params=_SC_PARAMS,
    scratch_types=[
        pltpu.VMEM((2, GRP, CH), jnp.int32),
        pltpu.VMEM((2, GRP, CH), jnp.int32),
        pltpu.VMEM((2, CH, D), jnp.float32),
        pltpu.VMEM_SHARED((NP, D), jnp.float32),
        pltpu.SemaphoreType.DMA,
        pltpu.SemaphoreType.DMA,
        pltpu.SemaphoreType.DMA,
        pltpu.SemaphoreType.DMA,
    ],
)
def _agg_pass(table_hbm, src_hbm, dst_hbm, zero_hbm, part_hbm,
              sbuf, dbuf, rows, acc, sg0, sg1, si0, si1):
    sg = (sg0, sg1)
    si = (si0, si1)
    c = lax.axis_index("c")
    s = lax.axis_index("s")
    wid = s * NC + c
    pltpu.sync_copy(zero_hbm, acc.at[pl.ds(s * RPT, RPT)])
    pltpu.sync_copy(src_hbm.at[wid, pl.ds(0, GRP)], sbuf.at[0])
    pltpu.sync_copy(dst_hbm.at[wid, pl.ds(0, GRP)], dbuf.at[0])
    plsc.subcore_barrier()
    pltpu.async_copy(table_hbm.at[sbuf.at[0, 0]], rows.at[0], sg[0])

    npair = NCH // (2 * GRP)

    def body(p, carry):
        base = p * 2 * GRP
        for ch in range(2 * GRP):
            b = ch % 2
            sl = ch // GRP
            k = ch % GRP
            if ch == 0:
                pltpu.async_copy(
                    src_hbm.at[wid, pl.ds(base + GRP, GRP)], sbuf.at[1], si[1])
                pltpu.async_copy(
                    dst_hbm.at[wid, pl.ds(base + GRP, GRP)], dbuf.at[1], si[1])
            if ch == GRP:
                @pl.when(p < npair - 1)
                def _():
                    pltpu.async_copy(
                        src_hbm.at[wid, pl.ds(base + 2 * GRP, GRP)],
                        sbuf.at[0], si[0])
                    pltpu.async_copy(
                        dst_hbm.at[wid, pl.ds(base + 2 * GRP, GRP)],
                        dbuf.at[0], si[0])
            if ch == GRP - 1:
                pltpu.make_async_copy(
                    src_hbm.at[wid, pl.ds(0, GRP)], sbuf.at[1], si[1]).wait()
                pltpu.make_async_copy(
                    dst_hbm.at[wid, pl.ds(0, GRP)], dbuf.at[1], si[1]).wait()
            if ch == 2 * GRP - 1:
                @pl.when(p < npair - 1)
                def _():
                    pltpu.make_async_copy(
                        src_hbm.at[wid, pl.ds(0, GRP)], sbuf.at[0],
                        si[0]).wait()
                    pltpu.make_async_copy(
                        dst_hbm.at[wid, pl.ds(0, GRP)], dbuf.at[0],
                        si[0]).wait()

            pltpu.make_async_copy(
                table_hbm.at[sbuf.at[sl, k]], rows.at[b], sg[b]).wait()
            nsl = ((ch + 1) % (2 * GRP)) // GRP
            nk = (ch + 1) % GRP
            if ch < 2 * GRP - 1:
                pltpu.async_copy(
                    table_hbm.at[sbuf.at[nsl, nk]], rows.at[1 - b], sg[1 - b])
            else:
                @pl.when(p < npair - 1)
                def _():
                    pltpu.async_copy(
                        table_hbm.at[sbuf.at[nsl, nk]], rows.at[1 - b],
                        sg[1 - b])
            pltpu.sync_copy(rows.at[b], acc.at[dbuf.at[sl, k]], add=True)
        return carry

    lax.fori_loop(0, npair, body, 0)
    plsc.subcore_barrier()
    pltpu.sync_copy(acc.at[pl.ds(s * RPT, RPT)],
                    part_hbm.at[c, pl.ds(s * RPT, RPT)])


def _final_body(p_ref, w_ref, deg_ref, b_ref, o_ref):
    a = p_ref[0] + p_ref[1]
    nr = lax.rsqrt(jnp.maximum(deg_ref[...], 1.0))
    o_ref[...] = (
        jnp.dot(a * nr, w_ref[...], preferred_element_type=jnp.float32,
                precision=lax.Precision.HIGHEST)
        + b_ref[...]
    )


def _final_pass(parts, W, deg_in_col, bias_row):
    blk = NP // 8
    return pl.pallas_call(
        _final_body,
        grid=(8,),
        in_specs=[
            pl.BlockSpec((NC, blk, D), lambda i: (0, i, 0)),
            pl.BlockSpec((D, D), lambda i: (0, 0)),
            pl.BlockSpec((blk, 1), lambda i: (i, 0)),
            pl.BlockSpec((1, D), lambda i: (0, 0)),
        ],
        out_specs=pl.BlockSpec((blk, D), lambda i: (i, 0)),
        out_shape=jax.ShapeDtypeStruct((NP, D), jnp.float32),
    )(parts, W, deg_in_col, bias_row)


def kernel(feat, edge_index, type_info, W, bias, weight_type):
    src = edge_index[0].astype(jnp.int32)
    dst = edge_index[1].astype(jnp.int32)
    pad = jnp.full((EPAD - E,), N, jnp.int32)
    src3 = jnp.concatenate([src, pad]).reshape(NW, NCH, CH)
    dst3 = jnp.concatenate([dst, pad]).reshape(NW, NCH, CH)

    feat_p = jnp.zeros((NP, D), jnp.float32).at[:N].set(feat)
    type_col = (
        jnp.zeros((NP,), jnp.int32).at[:N].set(type_info.astype(jnp.int32))
        .reshape(NP, 1)
    )
    wt_row = jnp.zeros((1, 128), jnp.float32).at[0, :4].set(weight_type)
    zero_blk = jnp.zeros((RPT, D), jnp.float32)

    hist = _deg_pass(src3, dst3)
    degs = _reduce_hist(hist)
    deg_out_col = degs[0].reshape(NP, 1)
    deg_in_col = degs[1].reshape(NP, 1)

    table = _scale_pass(feat_p, deg_out_col, type_col, wt_row)
    parts = _agg_pass(table, src3, dst3, zero_blk)
    out_p = _final_pass(parts, W, deg_in_col, bias.reshape(1, D))
    return out_p[:N]


# speedup vs baseline: 6.2115x; 1.4776x over previous
import functools

import numpy as np

import jax
import jax.numpy as jnp
from jax import lax
from jax.experimental import pallas as pl
from jax.experimental.pallas import tpu as pltpu
from jax.experimental.pallas import tpu_sc as plsc

N = 10000
E = 320000
D = 128
NP = 10240
NC = 2
NS = 16
NW = NC * NS
CH = 64
NCH = 160
EPW = NCH * CH
EPAD = NW * EPW
RPT = NP // NS
GRP = 8
NBG = 4
DH = D // 2

_MESH = plsc.VectorSubcoreMesh(core_axis_name="c", subcore_axis_name="s")
_SC_PARAMS = pltpu.CompilerParams(
    needs_layout_passes=False, use_tc_tiling_on_sc=False)

_PERM = np.empty((D,), dtype=np.int32)
for _q in range(4):
    for _j in range(16):
        _PERM[32 * _q + _j] = 32 * _q + 2 * _j
        _PERM[32 * _q + 16 + _j] = 32 * _q + 2 * _j + 1


@functools.partial(
    pl.kernel,
    out_type=jax.ShapeDtypeStruct((2, NW, NP), jnp.float32),
    mesh=_MESH,
    compiler_params=_SC_PARAMS,
    scratch_types=[
        pltpu.VMEM((NCH, CH), jnp.int32),
        pltpu.VMEM((NCH, CH), jnp.int32),
        pltpu.VMEM((NP,), jnp.float32),
        pltpu.VMEM((NP,), jnp.float32),
    ],
)
def _deg_pass(src_hbm, dst_hbm, hist_hbm, sidx, didx, hs, hd):
    c = lax.axis_index("c")
    s = lax.axis_index("s")
    wid = s * NC + c
    pltpu.sync_copy(src_hbm.at[wid], sidx)
    pltpu.sync_copy(dst_hbm.at[wid], didx)

    zeros = jnp.zeros((16,), jnp.float32)

    def zero_body(i, carry):
        hs[pl.ds(i * 16, 16)] = zeros
        hd[pl.ds(i * 16, 16)] = zeros
        return carry

    lax.fori_loop(0, NP // 16, zero_body, 0)

    ones = jnp.full((16,), 1.0, jnp.float32)

    def body(j, carry):
        for k in range(CH // 16):
            sv = sidx[j, pl.ds(k * 16, 16)]
            dv = didx[j, pl.ds(k * 16, 16)]
            plsc.addupdate_scatter(hs, [sv], ones)
            plsc.addupdate_scatter(hd, [dv], ones)
        return carry

    lax.fori_loop(0, NCH, body, 0)
    pltpu.sync_copy(hs, hist_hbm.at[0, wid])
    pltpu.sync_copy(hd, hist_hbm.at[1, wid])


def _reduce_body(hist_ref, deg_ref):
    deg_ref[...] = jnp.sum(hist_ref[...], axis=1)


def _reduce_hist(hist):
    return pl.pallas_call(
        _reduce_body,
        out_shape=jax.ShapeDtypeStruct((2, NP), jnp.float32),
    )(hist)


def _scale_body(feat_ref, deg_ref, type_ref, wt_ref, table_ref):
    nl = lax.rsqrt(jnp.maximum(deg_ref[...], 1.0))
    tc = type_ref[...]
    lanes = lax.broadcasted_iota(jnp.int32, (tc.shape[0], 128), 1)
    wtn = jnp.sum(
        jnp.where(tc == lanes, wt_ref[...], 0.0), axis=1, keepdims=True
    )
    table_ref[...] = (feat_ref[...] * (nl * wtn)).astype(jnp.bfloat16)


def _scale_pass(feat_p, deg_out_col, type_col, wt_row):
    blk = NP // 8
    return pl.pallas_call(
        _scale_body,
        grid=(8,),
        in_specs=[
            pl.BlockSpec((blk, D), lambda i: (i, 0)),
            pl.BlockSpec((blk, 1), lambda i: (i, 0)),
            pl.BlockSpec((blk, 1), lambda i: (i, 0)),
            pl.BlockSpec((1, 128), lambda i: (0, 0)),
        ],
        out_specs=pl.BlockSpec((blk, D), lambda i: (i, 0)),
        out_shape=jax.ShapeDtypeStruct((NP, D), jnp.bfloat16),
    )(feat_p, deg_out_col, type_col, wt_row)


@functools.partial(
    pl.kernel,
    out_type=jax.ShapeDtypeStruct((NC, NP, D), jnp.float32),
    mesh=_MESH,
    compiler_params=_SC_PARAMS,
    scratch_types=[
        pltpu.VMEM((2, GRP, CH), jnp.int32),
        pltpu.VMEM((2, GRP, CH), jnp.int32),
        pltpu.VMEM((NBG, CH, DH), jnp.int32),
        pltpu.VMEM((2, CH, D), jnp.float32),
        pltpu.VMEM_SHARED((NP, D), jnp.float32),
        pltpu.SemaphoreType.DMA,
        pltpu.SemaphoreType.DMA,
        pltpu.SemaphoreType.DMA,
        pltpu.SemaphoreType.DMA,
        pltpu.SemaphoreType.DMA,
        pltpu.SemaphoreType.DMA,
        pltpu.SemaphoreType.DMA,
        pltpu.SemaphoreType.DMA,
    ],
)
def _agg_pass(table_hbm, src_hbm, dst_hbm, zero_hbm, part_hbm,
              sbuf, dbuf, gring, fstage, acc,
              sg0, sg1, sg2, sg3, ss0, ss1, si0, si1):
    sg = (sg0, sg1, sg2, sg3)
    ss = (ss0, ss1)
    si = (si0, si1)
    c = lax.axis_index("c")
    s = lax.axis_index("s")
    wid = s * NC + c
    pltpu.sync_copy(zero_hbm, acc.at[pl.ds(s * RPT, RPT)])
    pltpu.sync_copy(src_hbm.at[wid, pl.ds(0, GRP)], sbuf.at[0])
    pltpu.sync_copy(dst_hbm.at[wid, pl.ds(0, GRP)], dbuf.at[0])
    plsc.subcore_barrier()
    for j in range(3):
        pltpu.async_copy(table_hbm.at[sbuf.at[0, j]], gring.at[j], sg[j])

    npair = NCH // (2 * GRP)

    def body(p, carry):
        base = p * 2 * GRP
        for ch in range(2 * GRP):
            gb = ch % NBG
            sb = ch % 2
            sl = ch // GRP
            k = ch % GRP
            if ch == 0:
                pltpu.async_copy(
                    src_hbm.at[wid, pl.ds(base + GRP, GRP)], sbuf.at[1], si[1])
                pltpu.async_copy(
                    dst_hbm.at[wid, pl.ds(base + GRP, GRP)], dbuf.at[1], si[1])
            if ch == GRP:
                @pl.when(p < npair - 1)
                def _():
                    pltpu.async_copy(
                        src_hbm.at[wid, pl.ds(base + 2 * GRP, GRP)],
                        sbuf.at[0], si[0])
                    pltpu.async_copy(
                        dst_hbm.at[wid, pl.ds(base + 2 * GRP, GRP)],
                        dbuf.at[0], si[0])
            if ch == GRP - 3:
                pltpu.make_async_copy(
                    src_hbm.at[wid, pl.ds(0, GRP)], sbuf.at[1], si[1]).wait()
                pltpu.make_async_copy(
                    dst_hbm.at[wid, pl.ds(0, GRP)], dbuf.at[1], si[1]).wait()
            if ch == 2 * GRP - 3:
                @pl.when(p < npair - 1)
                def _():
                    pltpu.make_async_copy(
                        src_hbm.at[wid, pl.ds(0, GRP)], sbuf.at[0],
                        si[0]).wait()
                    pltpu.make_async_copy(
                        dst_hbm.at[wid, pl.ds(0, GRP)], dbuf.at[0],
                        si[0]).wait()

            pltpu.make_async_copy(
                table_hbm.at[sbuf.at[sl, k]], gring.at[gb], sg[gb]).wait()

            if ch >= 2:
                pltpu.make_async_copy(
                    fstage.at[sb], acc.at[dbuf.at[sl, k]], ss[sb]).wait()
            else:
                @pl.when(p > 0)
                def _():
                    pltpu.make_async_copy(
                        fstage.at[sb], acc.at[dbuf.at[sl, k]], ss[sb]).wait()

            def conv_body(r, carry2, gb=gb, sb=sb):
                for q in range(4):
                    v = gring[gb, r, pl.ds(16 * q, 16)]
                    a, b2 = plsc.unpack(
                        plsc.bitcast(v, jnp.bfloat16),
                        format=plsc.PackFormat.INTERLEAVED)
                    fstage[sb, r, pl.ds(32 * q, 16)] = a
                    fstage[sb, r, pl.ds(32 * q + 16, 16)] = b2
                return carry2

            lax.fori_loop(0, CH, conv_body, 0)

            pltpu.async_copy(
                fstage.at[sb], acc.at[dbuf.at[sl, k]], ss[sb], add=True)

            c3 = ch + 3
            if c3 < 2 * GRP:
                nsl, nk = c3 // GRP, c3 % GRP
                pltpu.async_copy(
                    table_hbm.at[sbuf.at[nsl, nk]],
                    gring.at[c3 % NBG], sg[c3 % NBG])
            else:
                nsl, nk = 0, c3 - 2 * GRP

                @pl.when(p < npair - 1)
                def _():
                    pltpu.async_copy(
                        table_hbm.at[sbuf.at[nsl, nk]],
                        gring.at[c3 % NBG], sg[c3 % NBG])
        return carry

    lax.fori_loop(0, npair, body, 0)

    for b in range(2):
        pltpu.make_async_copy(
            fstage.at[b], acc.at[dbuf.at[1, GRP - 1]], ss[b]).wait()
    plsc.subcore_barrier()
    pltpu.sync_copy(acc.at[pl.ds(s * RPT, RPT)],
                    part_hbm.at[c, pl.ds(s * RPT, RPT)])


def _final_body(p_ref, w_ref, deg_ref, b_ref, o_ref):
    a = p_ref[0] + p_ref[1]
    nr = lax.rsqrt(jnp.maximum(deg_ref[...], 1.0))
    o_ref[...] = (
        jnp.dot(a * nr, w_ref[...], preferred_element_type=jnp.float32,
                precision=lax.Precision.HIGHEST)
        + b_ref[...]
    )


def _final_pass(parts, W_perm, deg_in_col, bias_row):
    blk = NP // 8
    return pl.pallas_call(
        _final_body,
        grid=(8,),
        in_specs=[
            pl.BlockSpec((NC, blk, D), lambda i: (0, i, 0)),
            pl.BlockSpec((D, D), lambda i: (0, 0)),
            pl.BlockSpec((blk, 1), lambda i: (i, 0)),
            pl.BlockSpec((1, D), lambda i: (0, 0)),
        ],
        out_specs=pl.BlockSpec((blk, D), lambda i: (i, 0)),
        out_shape=jax.ShapeDtypeStruct((NP, D), jnp.float32),
    )(parts, W_perm, deg_in_col, bias_row)


def kernel(feat, edge_index, type_info, W, bias, weight_type):
    src = edge_index[0].astype(jnp.int32)
    dst = edge_index[1].astype(jnp.int32)
    pad = jnp.full((EPAD - E,), N, jnp.int32)
    src3 = jnp.concatenate([src, pad]).reshape(NW, NCH, CH)
    dst3 = jnp.concatenate([dst, pad]).reshape(NW, NCH, CH)

    feat_p = jnp.zeros((NP, D), jnp.float32).at[:N].set(feat)
    type_col = (
        jnp.zeros((NP,), jnp.int32).at[:N].set(type_info.astype(jnp.int32))
        .reshape(NP, 1)
    )
    wt_row = jnp.zeros((1, 128), jnp.float32).at[0, :4].set(weight_type)
    zero_blk = jnp.zeros((RPT, D), jnp.float32)

    hist = _deg_pass(src3, dst3)
    degs = _reduce_hist(hist)
    deg_out_col = degs[0].reshape(NP, 1)
    deg_in_col = degs[1].reshape(NP, 1)

    table = _scale_pass(feat_p, deg_out_col, type_col, wt_row)
    table = lax.bitcast_convert_type(
        table.reshape(NP, DH, 2), jnp.int32)
    parts = _agg_pass(table, src3, dst3, zero_blk)
    W_perm = W[jnp.asarray(_PERM), :]
    out_p = _final_pass(parts, W_perm, deg_in_col, bias.reshape(1, D))
    return out_p[:N]
